# baseline (speedup 1.0000x reference)
import functools

import jax
import jax.numpy as jnp
from jax import lax
from jax.experimental import pallas as pl
from jax.experimental.pallas import tpu as pltpu
from jax.experimental.pallas import tpu_sc as plsc

N = 10000
E = 320000
DD = 64
H = 128

_NC = 2
_NS = 16
_NW = _NC * _NS
_EPW = E // _NW
_GW = 80
_GWIN = _EPW // _GW


def _gather_pre_body(pr_hbm, pc_hbm, row_hbm, col_hbm, g_hbm,
                     idxr_v, idxc_v, buf_v, sem):
    wid = lax.axis_index("s") * _NC + lax.axis_index("c")
    base = wid * _EPW

    def step(w, _):
        off = base + w * _GW
        pltpu.sync_copy(row_hbm.at[pl.ds(off, _GW)], idxr_v)
        pltpu.sync_copy(col_hbm.at[pl.ds(off, _GW)], idxc_v)
        pltpu.async_copy(pr_hbm.at[idxr_v], buf_v, sem).wait()
        pltpu.async_copy(pc_hbm.at[idxc_v], buf_v, sem, add=True).wait()
        pltpu.sync_copy(buf_v, g_hbm.at[pl.ds(off, _GW)])
        return _

    lax.fori_loop(0, _GWIN, step, 0)


def _gather_pre(P_r, P_c, row, col):
    mesh = plsc.VectorSubcoreMesh(core_axis_name="c", subcore_axis_name="s")
    return pl.kernel(
        _gather_pre_body,
        out_type=jax.ShapeDtypeStruct((E, H), jnp.float32),
        mesh=mesh,
        scratch_types=[
            pltpu.VMEM((_GW,), jnp.int32),
            pltpu.VMEM((_GW,), jnp.int32),
            pltpu.VMEM((_GW, H), jnp.float32),
            pltpu.SemaphoreType.DMA,
        ],
    )(P_r, P_c, row, col)


_TW = 128
_MW = 80
_MWIN = _EPW // _MW


_CH = 104


def _msgpass_body(t_hbm, s_hbm, row_hbm, col_hbm, zero_hbm, part_hbm,
                  idxr_v, idxc_v, rbuf, cbuf, sbuf, vbuf, acc_sp, sem):
    c = lax.axis_index("c")
    s = lax.axis_index("s")
    wid = s * _NC + c
    rpt = 624
    tail0 = rpt * _NS
    r0 = s * rpt
    pltpu.sync_copy(zero_hbm.at[pl.ds(r0, rpt)], acc_sp.at[pl.ds(r0, rpt)])

    @pl.when(s == _NS - 1)
    def _stage_tail():
        pltpu.sync_copy(zero_hbm.at[pl.ds(tail0, N - tail0)],
                        acc_sp.at[pl.ds(tail0, N - tail0)])

    plsc.subcore_barrier()
    base = wid * _EPW

    def step(w, carry):
        off = base + w * _MW
        pltpu.sync_copy(row_hbm.at[pl.ds(off, _MW)], idxr_v)
        pltpu.sync_copy(col_hbm.at[pl.ds(off, _MW)], idxc_v)
        cp1 = pltpu.async_copy(t_hbm.at[idxr_v], rbuf, sem)
        cp2 = pltpu.async_copy(t_hbm.at[idxc_v], cbuf, sem)
        pltpu.sync_copy(s_hbm.at[pl.ds(off, _MW)], sbuf)
        cp1.wait()
        cp2.wait()

        def edge(e, _):
            mr = rbuf[e, pl.ds(DD, 16)]
            mc = cbuf[e, pl.ds(DD, 16)]
            ok = (mr != 0.0) | (mc != 0.0)
            for j in range(DD // 16):
                g = cbuf[e, pl.ds(16 * j, 16)] - rbuf[e, pl.ds(16 * j, 16)]
                v = g * sbuf[e, pl.ds(16 * j, 16)]
                vbuf[e, pl.ds(16 * j, 16)] = jnp.where(ok, v, 0.0)
            return _

        lax.fori_loop(0, _MW, edge, 0)
        pltpu.sync_copy(vbuf, acc_sp.at[idxc_v], add=True)
        return carry

    lax.fori_loop(0, _MWIN, step, 0)
    plsc.subcore_barrier()
    pltpu.sync_copy(acc_sp.at[pl.ds(r0, rpt)], part_hbm.at[c, pl.ds(r0, rpt)])

    @pl.when(s == _NS - 1)
    def _write_tail():
        pltpu.sync_copy(acc_sp.at[pl.ds(tail0, N - tail0)],
                        part_hbm.at[c, pl.ds(tail0, N - tail0)])


def _msgpass(T, s_full, row, col):
    mesh = plsc.VectorSubcoreMesh(core_axis_name="c", subcore_axis_name="s")
    return pl.kernel(
        _msgpass_body,
        out_type=jax.ShapeDtypeStruct((_NC, N, DD), jnp.float32),
        mesh=mesh,
        scratch_types=[
            pltpu.VMEM((_MW,), jnp.int32),
            pltpu.VMEM((_MW,), jnp.int32),
            pltpu.VMEM((_MW, _TW), jnp.float32),
            pltpu.VMEM((_MW, _TW), jnp.float32),
            pltpu.VMEM((_MW, _TW), jnp.float32),
            pltpu.VMEM((_MW, DD), jnp.float32),
            pltpu.VMEM_SHARED((N, DD), jnp.float32),
            pltpu.SemaphoreType.DMA,
        ],
    )(T, s_full, row, col, jnp.zeros((N, DD), jnp.float32))


_SR = 312


def _sum_parts_body(part_hbm, out_hbm, b0, b1, pb):
    c = lax.axis_index("c")
    s = lax.axis_index("s")
    wid = s * _NC + c
    r0 = wid * _SR
    tail0 = _SR * _NW

    def zrow(i, _):
        for j in range(_TW // 16):
            pb[i, pl.ds(16 * j, 16)] = jnp.zeros((16,), jnp.float32)
        return _

    lax.fori_loop(0, _CH, zrow, 0)

    def chunk(base_row, nrows):
        pltpu.sync_copy(part_hbm.at[0, pl.ds(base_row, nrows)],
                        b0.at[pl.ds(0, nrows)])
        pltpu.sync_copy(part_hbm.at[1, pl.ds(base_row, nrows)],
                        b1.at[pl.ds(0, nrows)])

        def prow(r, _):
            for j in range(DD // 16):
                pb[r, pl.ds(16 * j, 16)] = (b0[r, pl.ds(16 * j, 16)] +
                                            b1[r, pl.ds(16 * j, 16)])
            return _

        lax.fori_loop(0, nrows, prow, 0)
        pltpu.sync_copy(pb.at[pl.ds(0, nrows)], out_hbm.at[pl.ds(base_row, nrows)])

    for i in range(_SR // _CH):
        chunk(r0 + i * _CH, _CH)

    @pl.when(wid == _NW - 1)
    def _tail():
        chunk(tail0, N - tail0)


def _sum_parts(part):
    mesh = plsc.VectorSubcoreMesh(core_axis_name="c", subcore_axis_name="s")
    return pl.kernel(
        _sum_parts_body,
        out_type=jax.ShapeDtypeStruct((N, _TW), jnp.float32),
        mesh=mesh,
        scratch_types=[
            pltpu.VMEM((_CH, DD), jnp.float32),
            pltpu.VMEM((_CH, DD), jnp.float32),
            pltpu.VMEM((_CH, _TW), jnp.float32),
        ],
    )(part)


def _mlp_body(g_ref, ea_ref, w1e_ref, b1_ref, w2t_ref, b2_ref, out_ref):
    pre = g_ref[...] + jnp.dot(ea_ref[...], w1e_ref[...],
                               preferred_element_type=jnp.float32) + b1_ref[...]
    h = jnp.maximum(pre, 0.0)
    s = jnp.dot(h, w2t_ref[...], preferred_element_type=jnp.float32) + b2_ref[...]
    nrm = jnp.sqrt(jnp.sum(s * s, axis=1, keepdims=True))
    s = s / nrm
    s = jnp.where(jnp.isnan(s), 0.0, s)
    out_ref[:, :DD] = s
    out_ref[:, DD:] = jnp.zeros_like(s)


def _edge_mlp(G, edge_attr, W1e_t, b1, W2_t, b2, block=2560):
    grid = E // block
    return pl.pallas_call(
        _mlp_body,
        grid=(grid,),
        in_specs=[
            pl.BlockSpec((block, H), lambda i: (i, 0)),
            pl.BlockSpec((block, 16), lambda i: (i, 0)),
            pl.BlockSpec((16, H), lambda i: (0, 0)),
            pl.BlockSpec((1, H), lambda i: (0, 0)),
            pl.BlockSpec((H, DD), lambda i: (0, 0)),
            pl.BlockSpec((1, DD), lambda i: (0, 0)),
        ],
        out_specs=pl.BlockSpec((block, _TW), lambda i: (i, 0)),
        out_shape=jax.ShapeDtypeStruct((E, _TW), jnp.float32),
    )(G, edge_attr, W1e_t, b1, W2_t, b2)


def kernel(x_s, x_d, edge_index, edge_attr, F0, F1, F2, W1, b1, W2, b2):
    row = edge_index[0]
    col = edge_index[1]
    W_sr = W1[:, 0:64]
    W_sc = W1[:, 64:128]
    W_dr = W1[:, 128:192]
    W_dc = W1[:, 192:256]
    W_e = W1[:, 256:272]
    P_r = x_s @ W_sr.T + x_d @ W_dr.T
    P_c = x_s @ W_sc.T + x_d @ W_dc.T

    G = _gather_pre(P_r, P_c, row, col)
    s_full = _edge_mlp(G, edge_attr, W_e.T, b1[None, :], W2.T, b2[None, :])

    s64 = s_full[:, :DD]
    out = x_d @ F0.T
    filters = (F1, F2)
    for k in range(2):
        mask = jnp.sum(out, axis=1) != 0
        em = (mask[row] | mask[col]).astype(out.dtype)
        grad = out[col] - out[row]
        shift = grad * s64 * em[:, None]
        scattered = jax.ops.segment_sum(shift, col, num_segments=N)
        out = out + scattered @ filters[k].T
    return out
